# baseline (speedup 1.0000x reference)
import jax
import jax.numpy as jnp
from jax.experimental import pallas as pl
from jax.experimental.pallas import tpu as pltpu

N = 4096
BS = 128
BA = 256
BC = 256
NS = N // BS
NA = N // BA
NC = N // BC
P_E = NS
P_C = NS + 1


def _gcn_kernel(adj_ref, x_ref, w0_ref, w1_ref, b0_ref, b1_ref, s_in_ref,
                wch_ref, wcs_ref, bc_ref, h_ref, y_ref,
                adjb_scr, s0_scr, s1_scr, d_scr, t_scr):
    i = pl.program_id(0)

    @pl.when(i < P_E)
    def _stream_wavefront():
        a = adj_ref[...]
        adjb_scr[pl.ds(i * BS, BS), :] = a.astype(jnp.bfloat16)
        d = jax.lax.rsqrt(1.0 + jnp.sum(a, axis=1, keepdims=True))
        d_scr[pl.ds(i * BS, BS), :] = d
        s0 = d * jnp.dot(x_ref[...], w0_ref[...],
                         preferred_element_type=jnp.float32)
        s0_scr[pl.ds(i * BS, BS), :] = s0.astype(jnp.bfloat16)

        @pl.when(i % 2 == 1)
        def _cells():
            k = i // 2

            t_scr[pl.ds(k * BA, BA), :] = jnp.dot(
                adjb_scr[pl.ds(k * BA, BA), pl.ds(0, BA)],
                s0_scr[pl.ds(0, BA), :], preferred_element_type=jnp.float32)

            def _row_cells(c, carry):
                acc = t_scr[pl.ds(k * BA, BA), :]
                acc += jnp.dot(adjb_scr[pl.ds(k * BA, BA), pl.ds(c * BA, BA)],
                               s0_scr[pl.ds(c * BA, BA), :],
                               preferred_element_type=jnp.float32)
                t_scr[pl.ds(k * BA, BA), :] = acc
                return carry

            jax.lax.fori_loop(1, k + 1, _row_cells, 0)

            def _col_cells(c, carry):
                acc = t_scr[pl.ds(c * BA, BA), :]
                acc += jnp.dot(adjb_scr[pl.ds(c * BA, BA), pl.ds(k * BA, BA)],
                               s0_scr[pl.ds(k * BA, BA), :],
                               preferred_element_type=jnp.float32)
                t_scr[pl.ds(c * BA, BA), :] = acc
                return carry

            jax.lax.fori_loop(0, k, _col_cells, 0)

    @pl.when(i == P_E)
    def _epilogue():
        def _chunk(c, carry):
            r = c * 512
            dd = d_scr[pl.ds(r, 512), :]
            h0 = dd * (t_scr[pl.ds(r, 512), :]
                       + s0_scr[pl.ds(r, 512), :].astype(jnp.float32)) \
                 + b0_ref[...]
            h0 = jnp.where(h0 >= 0, h0, 0.01 * h0)
            s1 = dd * jnp.dot(h0.astype(jnp.bfloat16), w1_ref[...],
                              preferred_element_type=jnp.float32)
            s1_scr[pl.ds(r, 512), :] = s1.astype(jnp.bfloat16)
            return carry

        jax.lax.fori_loop(0, 8, _chunk, 0)

    @pl.when(i > P_E)
    def _layer1():
        r = (i - P_C) * BC
        t = jnp.dot(adjb_scr[pl.ds(r, BC), :], s1_scr[...],
                    preferred_element_type=jnp.float32)
        own = s1_scr[pl.ds(r, BC), :].astype(jnp.float32)
        h = d_scr[pl.ds(r, BC), :] * (t + own) + b1_ref[...]
        h_ref[...] = h
        logits = (jnp.dot(h, wch_ref[...], preferred_element_type=jnp.float32)
                  + jnp.dot(s_in_ref[...], wcs_ref[...],
                            preferred_element_type=jnp.float32)
                  + bc_ref[...])
        m = jnp.max(logits, axis=1, keepdims=True)
        e = jnp.exp(logits - m)
        y_ref[...] = e / jnp.sum(e, axis=1, keepdims=True)


def kernel(x, adj, S, W0, b0, W1, b1, Wc, bc):
    in_dim = x.shape[1]
    hid = W0.shape[1]
    f_dim = W1.shape[1]
    s_dim = S.shape[1]
    c_dim = Wc.shape[1]

    def a_map(i):
        return (jnp.minimum(i, NS - 1), 0)

    def c_map(i):
        return (jnp.clip(i - P_C, 0, NC - 1), 0)

    h, y = pl.pallas_call(
        _gcn_kernel,
        grid=(NS + 1 + NC,),
        in_specs=[
            pl.BlockSpec((BS, N), a_map),
            pl.BlockSpec((BS, in_dim), a_map),
            pl.BlockSpec((in_dim, hid), lambda i: (0, 0)),
            pl.BlockSpec((hid, f_dim), lambda i: (0, 0)),
            pl.BlockSpec((1, hid), lambda i: (0, 0)),
            pl.BlockSpec((1, f_dim), lambda i: (0, 0)),
            pl.BlockSpec((BC, s_dim), c_map),
            pl.BlockSpec((f_dim, c_dim), lambda i: (0, 0)),
            pl.BlockSpec((s_dim, c_dim), lambda i: (0, 0)),
            pl.BlockSpec((1, c_dim), lambda i: (0, 0)),
        ],
        out_specs=[
            pl.BlockSpec((BC, f_dim), c_map),
            pl.BlockSpec((BC, c_dim), c_map),
        ],
        out_shape=[
            jax.ShapeDtypeStruct((N, f_dim), jnp.float32),
            jax.ShapeDtypeStruct((N, c_dim), jnp.float32),
        ],
        scratch_shapes=[
            pltpu.VMEM((N, N), jnp.bfloat16),
            pltpu.VMEM((N, hid), jnp.bfloat16),
            pltpu.VMEM((N, f_dim), jnp.bfloat16),
            pltpu.VMEM((N, 1), jnp.float32),
            pltpu.VMEM((N, hid), jnp.float32),
        ],
        compiler_params=pltpu.CompilerParams(
            dimension_semantics=("arbitrary",)),
    )(adj, x.astype(jnp.bfloat16), W0.astype(jnp.bfloat16), W1.astype(jnp.bfloat16),
      b0.reshape(1, hid), b1.reshape(1, f_dim), S,
      Wc[:f_dim], Wc[f_dim:], bc.reshape(1, c_dim))

    return (h, y)


# speedup vs baseline: 1.7621x; 1.7621x over previous
import jax
import jax.numpy as jnp
from jax.experimental import pallas as pl
from jax.experimental.pallas import tpu as pltpu

N = 4096
BS = 256
BB = 1024
BC = 512
NS = N // BS
NB = N // BB
NC = N // BC
P_P = NS
P_B = NS + 1
P_C = P_B + NB


def _gcn_kernel(adj_ref, x_ref, w0_ref, w1_ref, b0_ref, b1_ref, s_in_ref,
                wch_ref, wcs_ref, bc_ref, h_ref, y_ref,
                adjb_scr, s0_scr, s1_scr, d_scr):
    i = pl.program_id(0)

    @pl.when(i < P_P)
    def _stream():
        a = adj_ref[...]
        adjb_scr[pl.ds(i * BS, BS), :] = a.astype(jnp.bfloat16)
        d_scr[pl.ds(i * BS, BS), :] = jax.lax.rsqrt(
            1.0 + jnp.sum(a, axis=1, keepdims=True))

    @pl.when(i == P_P)
    def _prep():
        def _chunk(c, carry):
            r = c * BB
            s0 = d_scr[pl.ds(r, BB), :] * jnp.dot(
                x_ref[pl.ds(r, BB), :], w0_ref[...],
                preferred_element_type=jnp.float32)
            s0_scr[pl.ds(r, BB), :] = s0.astype(jnp.bfloat16)
            return carry

        jax.lax.fori_loop(0, NB, _chunk, 0)

    @pl.when(jnp.logical_and(i >= P_B, i < P_C))
    def _layer0():
        r = (i - P_B) * BB
        t = jnp.dot(adjb_scr[pl.ds(r, BB), :], s0_scr[...],
                    preferred_element_type=jnp.float32)
        own = s0_scr[pl.ds(r, BB), :].astype(jnp.float32)
        h0 = d_scr[pl.ds(r, BB), :] * (t + own) + b0_ref[...]
        h0 = jnp.where(h0 >= 0, h0, 0.01 * h0)
        s1 = d_scr[pl.ds(r, BB), :] * jnp.dot(
            h0, w1_ref[...], preferred_element_type=jnp.float32)
        s1_scr[pl.ds(r, BB), :] = s1.astype(jnp.bfloat16)

    @pl.when(i >= P_C)
    def _layer1():
        r = (i - P_C) * BC
        t = jnp.dot(adjb_scr[pl.ds(r, BC), :], s1_scr[...],
                    preferred_element_type=jnp.float32)
        own = s1_scr[pl.ds(r, BC), :].astype(jnp.float32)
        h = d_scr[pl.ds(r, BC), :] * (t + own) + b1_ref[...]
        h_ref[...] = h
        logits = (jnp.dot(h, wch_ref[...], preferred_element_type=jnp.float32)
                  + jnp.dot(s_in_ref[...], wcs_ref[...],
                            preferred_element_type=jnp.float32)
                  + bc_ref[...])
        m = jnp.max(logits, axis=1, keepdims=True)
        e = jnp.exp(logits - m)
        y_ref[...] = e / jnp.sum(e, axis=1, keepdims=True)


def kernel(x, adj, S, W0, b0, W1, b1, Wc, bc):
    in_dim = x.shape[1]
    hid = W0.shape[1]
    f_dim = W1.shape[1]
    s_dim = S.shape[1]
    c_dim = Wc.shape[1]

    def a_map(i):
        return (jnp.minimum(i, NS - 1), 0)

    def c_map(i):
        return (jnp.clip(i - P_C, 0, NC - 1), 0)

    h, y = pl.pallas_call(
        _gcn_kernel,
        grid=(NS + 1 + NB + NC,),
        in_specs=[
            pl.BlockSpec((BS, N), a_map),
            pl.BlockSpec((N, in_dim), lambda i: (0, 0)),
            pl.BlockSpec((in_dim, hid), lambda i: (0, 0)),
            pl.BlockSpec((hid, f_dim), lambda i: (0, 0)),
            pl.BlockSpec((1, hid), lambda i: (0, 0)),
            pl.BlockSpec((1, f_dim), lambda i: (0, 0)),
            pl.BlockSpec((BC, s_dim), c_map),
            pl.BlockSpec((f_dim, c_dim), lambda i: (0, 0)),
            pl.BlockSpec((s_dim, c_dim), lambda i: (0, 0)),
            pl.BlockSpec((1, c_dim), lambda i: (0, 0)),
        ],
        out_specs=[
            pl.BlockSpec((BC, f_dim), c_map),
            pl.BlockSpec((BC, c_dim), c_map),
        ],
        out_shape=[
            jax.ShapeDtypeStruct((N, f_dim), jnp.float32),
            jax.ShapeDtypeStruct((N, c_dim), jnp.float32),
        ],
        scratch_shapes=[
            pltpu.VMEM((N, N), jnp.bfloat16),
            pltpu.VMEM((N, hid), jnp.bfloat16),
            pltpu.VMEM((N, f_dim), jnp.bfloat16),
            pltpu.VMEM((N, 1), jnp.float32),
        ],
        compiler_params=pltpu.CompilerParams(
            dimension_semantics=("arbitrary",)),
    )(adj, x.astype(jnp.bfloat16), W0.astype(jnp.bfloat16), W1,
      b0.reshape(1, hid), b1.reshape(1, f_dim), S,
      Wc[:f_dim], Wc[f_dim:], bc.reshape(1, c_dim))

    return (h, y)


# speedup vs baseline: 1.8036x; 1.0236x over previous
import jax
import jax.numpy as jnp
from jax.experimental import pallas as pl
from jax.experimental.pallas import tpu as pltpu

N = 4096
BS = 256
BB = 1024
BC = 1024
NS = N // BS
NB = N // BB
NC = N // BC
P_P = NS
P_B = NS + NB
P_C = P_B + NB


def _gcn_kernel(adj_ref, x_ref, w0_ref, w1_ref, b0_ref, b1_ref, s_in_ref,
                wch_ref, wcs_ref, bc_ref, h_ref, y_ref,
                adjb_scr, s0_scr, s1_scr, d_scr):
    i = pl.program_id(0)

    @pl.when(i < P_P)
    def _stream():
        a = adj_ref[...]
        adjb_scr[pl.ds(i * BS, BS), :] = a.astype(jnp.bfloat16)
        d_scr[pl.ds(i * BS, BS), :] = jax.lax.rsqrt(
            1.0 + jnp.sum(a, axis=1, keepdims=True))

    @pl.when(jnp.logical_and(i >= P_P, i < P_B))
    def _prep():
        r = (i - P_P) * BB
        s0 = d_scr[pl.ds(r, BB), :] * jnp.dot(
            x_ref[...], w0_ref[...], preferred_element_type=jnp.float32)
        s0_scr[pl.ds(r, BB), :] = s0.astype(jnp.bfloat16)

    @pl.when(jnp.logical_and(i >= P_B, i < P_C))
    def _layer0():
        r = (i - P_B) * BB
        t = jnp.dot(adjb_scr[pl.ds(r, BB), :], s0_scr[...],
                    preferred_element_type=jnp.float32)
        own = s0_scr[pl.ds(r, BB), :].astype(jnp.float32)
        h0 = d_scr[pl.ds(r, BB), :] * (t + own) + b0_ref[...]
        h0 = jnp.where(h0 >= 0, h0, 0.01 * h0)
        s1 = d_scr[pl.ds(r, BB), :] * jnp.dot(
            h0, w1_ref[...], preferred_element_type=jnp.float32)
        s1_scr[pl.ds(r, BB), :] = s1.astype(jnp.bfloat16)

    @pl.when(i >= P_C)
    def _layer1():
        r = (i - P_C) * BC
        t = jnp.dot(adjb_scr[pl.ds(r, BC), :], s1_scr[...],
                    preferred_element_type=jnp.float32)
        own = s1_scr[pl.ds(r, BC), :].astype(jnp.float32)
        h = d_scr[pl.ds(r, BC), :] * (t + own) + b1_ref[...]
        h_ref[...] = h
        logits = (jnp.dot(h, wch_ref[...], preferred_element_type=jnp.float32)
                  + jnp.dot(s_in_ref[...], wcs_ref[...],
                            preferred_element_type=jnp.float32)
                  + bc_ref[...])
        m = jnp.max(logits, axis=1, keepdims=True)
        e = jnp.exp(logits - m)
        y_ref[...] = e / jnp.sum(e, axis=1, keepdims=True)


def kernel(x, adj, S, W0, b0, W1, b1, Wc, bc):
    in_dim = x.shape[1]
    hid = W0.shape[1]
    f_dim = W1.shape[1]
    s_dim = S.shape[1]
    c_dim = Wc.shape[1]

    def a_map(i):
        return (jnp.minimum(i, NS - 1), 0)

    def x_map(i):
        return (jnp.clip(i - P_P, 0, NB - 1), 0)

    def c_map(i):
        return (jnp.clip(i - P_C, 0, NC - 1), 0)

    h, y = pl.pallas_call(
        _gcn_kernel,
        grid=(NS + NB + NB + NC,),
        in_specs=[
            pl.BlockSpec((BS, N), a_map),
            pl.BlockSpec((BB, in_dim), x_map),
            pl.BlockSpec((in_dim, hid), lambda i: (0, 0)),
            pl.BlockSpec((hid, f_dim), lambda i: (0, 0)),
            pl.BlockSpec((1, hid), lambda i: (0, 0)),
            pl.BlockSpec((1, f_dim), lambda i: (0, 0)),
            pl.BlockSpec((BC, s_dim), c_map),
            pl.BlockSpec((f_dim, c_dim), lambda i: (0, 0)),
            pl.BlockSpec((s_dim, c_dim), lambda i: (0, 0)),
            pl.BlockSpec((1, c_dim), lambda i: (0, 0)),
        ],
        out_specs=[
            pl.BlockSpec((BC, f_dim), c_map),
            pl.BlockSpec((BC, c_dim), c_map),
        ],
        out_shape=[
            jax.ShapeDtypeStruct((N, f_dim), jnp.float32),
            jax.ShapeDtypeStruct((N, c_dim), jnp.float32),
        ],
        scratch_shapes=[
            pltpu.VMEM((N, N), jnp.bfloat16),
            pltpu.VMEM((N, hid), jnp.bfloat16),
            pltpu.VMEM((N, f_dim), jnp.bfloat16),
            pltpu.VMEM((N, 1), jnp.float32),
        ],
        compiler_params=pltpu.CompilerParams(
            dimension_semantics=("arbitrary",)),
    )(adj, x.astype(jnp.bfloat16), W0.astype(jnp.bfloat16), W1,
      b0.reshape(1, hid), b1.reshape(1, f_dim), S,
      Wc[:f_dim], Wc[f_dim:], bc.reshape(1, c_dim))

    return (h, y)


# speedup vs baseline: 2.0460x; 1.1344x over previous
import jax
import jax.numpy as jnp
from jax.experimental import pallas as pl
from jax.experimental.pallas import tpu as pltpu

N = 4096
BA = 256
BB = 1024
NA = N // BA
NB = N // BB
P_B = NA
P_C = NA + NB


def _gcn_kernel(adj_ref, x_ref, w0_ref, w1_ref, b0_ref, b1_ref, s_in_ref,
                wch_ref, wcs_ref, bc_ref, h_ref, y_ref,
                adjb_scr, s0_scr, s1_scr, d_scr):
    i = pl.program_id(0)

    @pl.when(i < P_B)
    def _phase_a():
        a = adj_ref[...]
        adjb_scr[pl.ds(i * BA, BA), :] = a.astype(jnp.bfloat16)
        d = jax.lax.rsqrt(1.0 + jnp.sum(a, axis=1, keepdims=True))
        d_scr[pl.ds(i * BA, BA), :] = d
        s0 = d * jnp.dot(x_ref[...], w0_ref[...],
                         preferred_element_type=jnp.float32)
        s0_scr[pl.ds(i * BA, BA), :] = s0.astype(jnp.bfloat16)

    @pl.when(jnp.logical_and(i >= P_B, i < P_C))
    def _phase_b():
        r = (i - P_B) * BB
        t = jnp.dot(adjb_scr[pl.ds(r, BB), :], s0_scr[...],
                    preferred_element_type=jnp.float32)
        own = s0_scr[pl.ds(r, BB), :].astype(jnp.float32)
        h0 = d_scr[pl.ds(r, BB), :] * (t + own) + b0_ref[...]
        h0 = jnp.where(h0 >= 0, h0, 0.01 * h0)
        s1 = d_scr[pl.ds(r, BB), :] * jnp.dot(
            h0, w1_ref[...], preferred_element_type=jnp.float32)
        s1_scr[pl.ds(r, BB), :] = s1.astype(jnp.bfloat16)

    @pl.when(i >= P_C)
    def _phase_c():
        r = (i - P_C) * BB
        t = jnp.dot(adjb_scr[pl.ds(r, BB), :], s1_scr[...],
                    preferred_element_type=jnp.float32)
        own = s1_scr[pl.ds(r, BB), :].astype(jnp.float32)
        h = d_scr[pl.ds(r, BB), :] * (t + own) + b1_ref[...]
        h_ref[...] = h
        logits = (jnp.dot(h, wch_ref[...], preferred_element_type=jnp.float32)
                  + jnp.dot(s_in_ref[...], wcs_ref[...],
                            preferred_element_type=jnp.float32)
                  + bc_ref[...])
        m = jnp.max(logits, axis=1, keepdims=True)
        e = jnp.exp(logits - m)
        y_ref[...] = e / jnp.sum(e, axis=1, keepdims=True)


def kernel(x, adj, S, W0, b0, W1, b1, Wc, bc):
    in_dim = x.shape[1]
    hid = W0.shape[1]
    f_dim = W1.shape[1]
    s_dim = S.shape[1]
    c_dim = Wc.shape[1]

    def a_map(i):
        return (jnp.minimum(i, NA - 1), 0)

    def c_map(i):
        return (jnp.clip(i - P_C, 0, NB - 1), 0)

    h, y = pl.pallas_call(
        _gcn_kernel,
        grid=(NA + NB + NB,),
        in_specs=[
            pl.BlockSpec((BA, N), a_map),
            pl.BlockSpec((BA, in_dim), a_map),
            pl.BlockSpec((in_dim, hid), lambda i: (0, 0)),
            pl.BlockSpec((hid, f_dim), lambda i: (0, 0)),
            pl.BlockSpec((1, hid), lambda i: (0, 0)),
            pl.BlockSpec((1, f_dim), lambda i: (0, 0)),
            pl.BlockSpec((BB, s_dim), c_map),
            pl.BlockSpec((f_dim, c_dim), lambda i: (0, 0)),
            pl.BlockSpec((s_dim, c_dim), lambda i: (0, 0)),
            pl.BlockSpec((1, c_dim), lambda i: (0, 0)),
        ],
        out_specs=[
            pl.BlockSpec((BB, f_dim), c_map),
            pl.BlockSpec((BB, c_dim), c_map),
        ],
        out_shape=[
            jax.ShapeDtypeStruct((N, f_dim), jnp.float32),
            jax.ShapeDtypeStruct((N, c_dim), jnp.float32),
        ],
        scratch_shapes=[
            pltpu.VMEM((N, N), jnp.bfloat16),
            pltpu.VMEM((N, hid), jnp.bfloat16),
            pltpu.VMEM((N, f_dim), jnp.bfloat16),
            pltpu.VMEM((N, 1), jnp.float32),
        ],
        compiler_params=pltpu.CompilerParams(
            dimension_semantics=("arbitrary",)),
    )(adj, x, W0, W1, b0.reshape(1, hid), b1.reshape(1, f_dim), S,
      Wc[:f_dim], Wc[f_dim:], bc.reshape(1, c_dim))

    return (h, y)
